# baseline (speedup 1.0000x reference)
import functools
import numpy as np
import jax
import jax.numpy as jnp
from jax import lax
from jax.experimental import pallas as pl
from jax.experimental.pallas import tpu as pltpu
from jax.experimental.pallas import tpu_sc as plsc

N = 10000
E = 320000
D = 128
NC = 2
NS = 16
NW = NC * NS
EPW = E // NW
NSP = 10112
ZPT = 632
ZR = 320
RPT = 624


def _gather16(x, idx):
    dn = lax.GatherDimensionNumbers(
        offset_dims=(), collapsed_slice_dims=(0,), start_index_map=(0,))
    return lax.gather(x, idx[:, None], dn, (1,),
                      mode=lax.GatherScatterMode.PROMISE_IN_BOUNDS)


def _lane_perm(x, stride):
    return _gather16(x, lax.iota(jnp.int32, 16) ^ stride)


def _group_sum(t, width):
    s = 1
    while s < width:
        t = t + _lane_perm(t, s)
        s *= 2
    return t


def _make_edge_call(F, group, EB):
    VPR = F // 16
    NCHUNK = EPW // EB

    mesh = plsc.VectorSubcoreMesh(core_axis_name="c", subcore_axis_name="s")

    @functools.partial(
        pl.kernel,
        out_type=(pltpu.HBM((NC, N, F), jnp.float32),
                  pltpu.HBM((NC, N, 16), jnp.float32)),
        mesh=mesh,
        compiler_params=pltpu.CompilerParams(use_tc_tiling_on_sc=False),
        scratch_types=[
            pltpu.VMEM((EB,), jnp.int32),
            pltpu.VMEM((EB,), jnp.int32),
            pltpu.VMEM((EB,), jnp.int32),
            pltpu.VMEM((EB,), jnp.int32),
            pltpu.VMEM((EB, F), jnp.float32),
            pltpu.VMEM((EB, F), jnp.float32),
            pltpu.VMEM((EB, F), jnp.float32),
            pltpu.VMEM((EB, F), jnp.float32),
            pltpu.VMEM((EB, F), jnp.float32),
            pltpu.VMEM((EB, F), jnp.float32),
            pltpu.VMEM((EB, F), jnp.float32),
            pltpu.VMEM((EB, 16), jnp.float32),
            pltpu.VMEM((ZR, F), jnp.float32),
            pltpu.VMEM((ZR, 16), jnp.float32),
            pltpu.VMEM((F,), jnp.float32),
            pltpu.SemaphoreType.DMA,
            pltpu.SemaphoreType.DMA,
            pltpu.VMEM_SHARED((NSP, F), jnp.float32),
            pltpu.VMEM_SHARED((NSP, 16), jnp.float32),
        ],
    )
    def edge_kernel(q_hbm, k_hbm, v_hbm, src_hbm, dst_hbm, mv_hbm,
                    outm_hbm, outd_hbm,
                    sv0, sv1, dv0, dv1, qv0, qv1, kv0, kv1, vv0, vv1,
                    mbuf, dbuf, zbuf, zbufd, mv, sem_i, sem_r,
                    accm_sh, accd_sh):
        svs, dvs = (sv0, sv1), (dv0, dv1)
        qvs, kvs, vvs = (qv0, qv1), (kv0, kv1), (vv0, vv1)
        HB = EB // 2

        def split_gather(tab, idxr, dstbuf):
            pltpu.async_copy(tab.at[idxr.at[pl.ds(0, HB)]],
                             dstbuf.at[pl.ds(0, HB)], sem_r)
            pltpu.async_copy(tab.at[idxr.at[pl.ds(HB, HB)]],
                             dstbuf.at[pl.ds(HB, HB)], sem_r)
        cid = lax.axis_index("c")
        sid = lax.axis_index("s")
        wid = sid * NC + cid

        zvec = jnp.zeros((16,), jnp.float32)
        lane = lax.iota(jnp.int32, 16)

        def zrow(i, _):
            for j in range(F // 16):
                zbuf[i, pl.ds(16 * j, 16)] = zvec
            zbufd[i, pl.ds(0, 16)] = zvec
            return 0
        lax.fori_loop(0, ZR, zrow, 0)

        r0 = sid * ZPT
        pltpu.sync_copy(zbuf.at[pl.ds(0, 312)], accm_sh.at[pl.ds(r0, 312)])
        pltpu.sync_copy(zbuf, accm_sh.at[pl.ds(r0 + 312, 320)])
        pltpu.sync_copy(zbufd.at[pl.ds(0, 312)], accd_sh.at[pl.ds(r0, 312)])
        pltpu.sync_copy(zbufd, accd_sh.at[pl.ds(r0 + 312, 320)])
        pltpu.sync_copy(mv_hbm, mv)
        plsc.subcore_barrier()

        base = wid * EPW

        mvj = [mv[pl.ds(16 * j, 16)] for j in range(VPR)]
        pick_idx = (lane & 1) * 8
        hsels = [(lane >> 1) == j for j in range(VPR)]
        lane0 = lane == 0

        def body(c, b, do_idx, do_gather):
            pltpu.make_async_copy(q_hbm.at[pl.ds(0, EB)], qvs[b],
                                  sem_r).wait()
            pltpu.make_async_copy(k_hbm.at[pl.ds(0, EB)], kvs[b],
                                  sem_r).wait()
            pltpu.make_async_copy(v_hbm.at[pl.ds(0, EB)], vvs[b],
                                  sem_r).wait()
            nb = 1 - b
            if do_gather:
                pltpu.make_async_copy(src_hbm.at[pl.ds(0, EB)], svs[nb],
                                      sem_i).wait()
                pltpu.make_async_copy(dst_hbm.at[pl.ds(0, EB)], dvs[nb],
                                      sem_i).wait()
                split_gather(q_hbm, dvs[nb], qvs[nb])
                split_gather(k_hbm, svs[nb], kvs[nb])
                split_gather(v_hbm, svs[nb], vvs[nb])

            off = base + c * EB
            qv, kv, vv = qvs[b], kvs[b], vvs[b]

            def edge_body(e2, _):
                for ee in range(2):
                    e = e2 * 2 + ee
                    den = zvec
                    for j in range(VPR):
                        qr = qv[e, pl.ds(16 * j, 16)]
                        kr = kv[e, pl.ds(16 * j, 16)]
                        vr = vv[e, pl.ds(16 * j, 16)]
                        t = _group_sum(qr * kr, group)
                        ex = jnp.exp(t - mvj[j])
                        mbuf[e, pl.ds(16 * j, 16)] = vr * ex
                        if group == 16:
                            den = jnp.where(lane0, ex, 0.0)
                        else:
                            picked = _gather16(ex, pick_idx)
                            den = den + jnp.where(hsels[j], picked, 0.0)
                    dbuf[e, pl.ds(0, 16)] = den
                return 0
            lax.fori_loop(0, EB // 2, edge_body, 0)

            pltpu.sync_copy(mbuf, accm_sh.at[dvs[b]], add=True)
            pltpu.sync_copy(dbuf, accd_sh.at[dvs[b]], add=True)

            if do_idx:
                off2 = base + (c + 2) * EB
                pltpu.async_copy(src_hbm.at[pl.ds(off2, EB)], svs[b], sem_i)
                pltpu.async_copy(dst_hbm.at[pl.ds(off2, EB)], dvs[b], sem_i)

        pltpu.sync_copy(src_hbm.at[pl.ds(base, EB)], sv0)
        pltpu.sync_copy(dst_hbm.at[pl.ds(base, EB)], dv0)
        pltpu.async_copy(src_hbm.at[pl.ds(base + EB, EB)], sv1, sem_i)
        pltpu.async_copy(dst_hbm.at[pl.ds(base + EB, EB)], dv1, sem_i)
        split_gather(q_hbm, dv0, qv0)
        split_gather(k_hbm, sv0, kv0)
        split_gather(v_hbm, sv0, vv0)

        def pair_body(c2, _):
            c = c2 * 2
            body(c, 0, True, True)
            body(c + 1, 1, True, True)
            return 0
        if NCHUNK % 2 == 0:
            lax.fori_loop(0, NCHUNK // 2 - 1, pair_body, 0)
        else:
            lax.fori_loop(0, (NCHUNK - 3) // 2, pair_body, 0)
            body(NCHUNK - 3, 0, True, True)
        body(NCHUNK - 2, (NCHUNK - 2) % 2, False, True)
        body(NCHUNK - 1, (NCHUNK - 1) % 2, False, False)

        plsc.subcore_barrier()
        w0 = sid * RPT
        pltpu.sync_copy(accm_sh.at[pl.ds(w0, RPT)],
                        outm_hbm.at[cid, pl.ds(w0, RPT)])
        pltpu.sync_copy(accd_sh.at[pl.ds(w0, RPT)],
                        outd_hbm.at[cid, pl.ds(w0, RPT)])

        @pl.when(sid == NS - 1)
        def _tail():
            pltpu.sync_copy(accm_sh.at[pl.ds(NS * RPT, N - NS * RPT)],
                            outm_hbm.at[cid, pl.ds(NS * RPT, N - NS * RPT)])
            pltpu.sync_copy(accd_sh.at[pl.ds(NS * RPT, N - NS * RPT)],
                            outd_hbm.at[cid, pl.ds(NS * RPT, N - NS * RPT)])

    return edge_kernel


_edge_call_64 = _make_edge_call(64, 8, 80)
_edge_call_16 = _make_edge_call(16, 16, 80)

BN = 1000
GRID = N // BN


def _dense1_body(x_ref, wq_ref, wk_ref, wv_ref, ws_ref,
                 bq_ref, bk_ref, bv_ref, bs_ref, selq_ref, selk_ref,
                 t1_ref, t2_ref,
                 q_ref, k_ref, v_ref, s_ref, mv_ref, mn_scr):
    i = pl.program_id(0)
    xb = x_ref[...]
    dn = (((1,), (1,)), ((), ()))
    q = (lax.dot_general(xb, wq_ref[...], dn,
                         preferred_element_type=jnp.float32)
         + bq_ref[...]) * np.float32(1.0 / np.sqrt(8.0))
    k = lax.dot_general(xb, wk_ref[...], dn,
                        preferred_element_type=jnp.float32) + bk_ref[...]
    v = lax.dot_general(xb, wv_ref[...], dn,
                        preferred_element_type=jnp.float32) + bv_ref[...]
    s = lax.dot_general(xb, ws_ref[...], dn,
                        preferred_element_type=jnp.float32) + bs_ref[...]
    q_ref[...] = q
    k_ref[...] = k
    v_ref[...] = v
    s_ref[...] = s
    dn2 = (((1,), (0,)), ((), ()))
    qn = lax.dot_general(q * q, selq_ref[...], dn2,
                         preferred_element_type=jnp.float32)
    kn = lax.dot_general(k * k, selk_ref[...], dn2,
                         preferred_element_type=jnp.float32)
    cur = (jnp.max(qn, axis=0, keepdims=True)
           + jnp.max(kn, axis=0, keepdims=True))

    @pl.when(i == 0)
    def _init():
        mn_scr[...] = cur

    @pl.when(i > 0)
    def _acc():
        mn_scr[...] = jnp.maximum(mn_scr[...], cur)

    @pl.when(i == GRID - 1)
    def _fin():
        mn = mn_scr[...]
        mv_ref[...] = jnp.sqrt(
            lax.dot_general(mn, t1_ref[...], dn2,
                            preferred_element_type=jnp.float32)
            * lax.dot_general(mn, t2_ref[...], dn2,
                              preferred_element_type=jnp.float32))[None]


def _dense2_body(pm_ref, pd_ref, s1_ref, wq_ref, wk_ref, wv_ref, ws_ref,
                 bq_ref, bk_ref, bv_ref, bs_ref, selq_ref, selk_ref, rep_ref,
                 t1_ref, t2_ref,
                 q_ref, k_ref, v_ref, s_ref, mv_ref, mn_scr):
    i = pl.program_id(0)
    pm = pm_ref[...]
    pd = pd_ref[...]
    acc = pm[0] + pm[1]
    den8 = pd[0] + pd[1]
    dn2 = (((1,), (0,)), ((), ()))
    den = lax.dot_general(den8, rep_ref[...], dn2,
                          preferred_element_type=jnp.float32)
    o = acc / (den + 1e-16) + s1_ref[...]
    h = jnp.where(o > 0, o, jnp.exp(o) - 1.0)
    dn = (((1,), (1,)), ((), ()))
    q = (lax.dot_general(h, wq_ref[...], dn,
                         preferred_element_type=jnp.float32)
         + bq_ref[...]) * np.float32(0.25)
    k = lax.dot_general(h, wk_ref[...], dn,
                        preferred_element_type=jnp.float32) + bk_ref[...]
    v = lax.dot_general(h, wv_ref[...], dn,
                        preferred_element_type=jnp.float32) + bv_ref[...]
    s = lax.dot_general(h, ws_ref[...], dn,
                        preferred_element_type=jnp.float32) + bs_ref[...]
    q_ref[...] = q
    k_ref[...] = k
    v_ref[...] = v
    s_ref[...] = s
    qn = lax.dot_general(q * q, selq_ref[...], dn2,
                         preferred_element_type=jnp.float32)
    kn = lax.dot_general(k * k, selk_ref[...], dn2,
                         preferred_element_type=jnp.float32)
    cur = (jnp.max(qn, axis=0, keepdims=True)
           + jnp.max(kn, axis=0, keepdims=True))

    @pl.when(i == 0)
    def _init():
        mn_scr[...] = cur

    @pl.when(i > 0)
    def _acc():
        mn_scr[...] = jnp.maximum(mn_scr[...], cur)

    @pl.when(i == GRID - 1)
    def _fin():
        mn = mn_scr[...]
        mv_ref[...] = jnp.sqrt(
            lax.dot_general(mn, t1_ref[...], dn2,
                            preferred_element_type=jnp.float32)
            * lax.dot_general(mn, t2_ref[...], dn2,
                              preferred_element_type=jnp.float32))[None]


def _final_body(pm_ref, pd_ref, s2_ref, o_ref):
    pm = pm_ref[...]
    pd = pd_ref[...]
    acc = pm[0] + pm[1]
    den = (pd[0] + pd[1])[:, 0:1]
    o = acc / (den + 1e-16) + s2_ref[...]
    m = jnp.max(o, axis=1, keepdims=True)
    ex = jnp.exp(o - m)
    o_ref[...] = o - m - jnp.log(jnp.sum(ex, axis=1, keepdims=True))


def _sel_mats(F, heads):
    c = F // heads
    sq = np.zeros((F, 128), np.float32)
    sk = np.zeros((F, 128), np.float32)
    for h in range(heads):
        sq[h * c:(h + 1) * c, h] = 1.0
        sk[h * c:(h + 1) * c, heads + h] = 1.0
    return sq, sk


_SELQ1, _SELK1 = _sel_mats(64, 8)
_SELQ2, _SELK2 = _sel_mats(16, 1)


def _t_mats(F, heads):
    c = F // heads
    t1 = np.zeros((128, F), np.float32)
    t2 = np.zeros((128, F), np.float32)
    for h in range(heads):
        t1[h, h * c:(h + 1) * c] = 1.0
        t2[heads + h, h * c:(h + 1) * c] = 1.0
    return t1, t2


_T11, _T21 = _t_mats(64, 8)
_T12, _T22 = _t_mats(16, 1)

_REP = np.zeros((16, 64), np.float32)
for _h in range(8):
    _REP[_h, _h * 8:(_h + 1) * 8] = 1.0


def _row_spec(w):
    return pl.BlockSpec((BN, w), lambda i: (i, 0))


def _full_spec(shape):
    nd = len(shape)
    return pl.BlockSpec(shape, lambda i, _n=nd: (0,) * _n)


def kernel(x, edge_index, Wq1, bq1, Wk1, bk1, Wv1, bv1, Ws1, bs1,
           Wq2, bq2, Wk2, bk2, Wv2, bv2, Ws2, bs2):
    src_p = edge_index[0].astype(jnp.int32)
    dst_p = edge_index[1].astype(jnp.int32)

    dense1 = pl.pallas_call(
        _dense1_body,
        grid=(GRID,),
        in_specs=[
            _row_spec(D),
            _full_spec((64, D)), _full_spec((64, D)),
            _full_spec((64, D)), _full_spec((64, D)),
            _full_spec((1, 64)), _full_spec((1, 64)),
            _full_spec((1, 64)), _full_spec((1, 64)),
            _full_spec((64, 128)), _full_spec((64, 128)),
            _full_spec((128, 64)), _full_spec((128, 64)),
        ],
        out_specs=[
            _row_spec(64), _row_spec(64), _row_spec(64), _row_spec(64),
            pl.BlockSpec((1, 1, 64), lambda i: (0, 0, 0)),
        ],
        out_shape=[
            jax.ShapeDtypeStruct((N, 64), jnp.float32),
            jax.ShapeDtypeStruct((N, 64), jnp.float32),
            jax.ShapeDtypeStruct((N, 64), jnp.float32),
            jax.ShapeDtypeStruct((N, 64), jnp.float32),
            jax.ShapeDtypeStruct((1, 1, 64), jnp.float32),
        ],
        scratch_shapes=[pltpu.VMEM((1, 128), jnp.float32)],
    )
    q1, k1, v1, s1, mv1 = dense1(
        x, Wq1, Wk1, Wv1, Ws1,
        bq1.reshape(1, 64), bk1.reshape(1, 64),
        bv1.reshape(1, 64), bs1.reshape(1, 64),
        _SELQ1, _SELK1, _T11, _T21)

    pm1, pd1 = _edge_call_64(q1, k1, v1, src_p, dst_p, mv1.reshape(64))

    dense2 = pl.pallas_call(
        _dense2_body,
        grid=(GRID,),
        in_specs=[
            pl.BlockSpec((NC, BN, 64), lambda i: (0, i, 0)),
            pl.BlockSpec((NC, BN, 16), lambda i: (0, i, 0)),
            _row_spec(64),
            _full_spec((16, 64)), _full_spec((16, 64)),
            _full_spec((16, 64)), _full_spec((16, 64)),
            _full_spec((1, 16)), _full_spec((1, 16)),
            _full_spec((1, 16)), _full_spec((1, 16)),
            _full_spec((16, 128)), _full_spec((16, 128)), _full_spec((16, 64)),
            _full_spec((128, 16)), _full_spec((128, 16)),
        ],
        out_specs=[
            _row_spec(16), _row_spec(16), _row_spec(16), _row_spec(16),
            pl.BlockSpec((1, 1, 16), lambda i: (0, 0, 0)),
        ],
        out_shape=[
            jax.ShapeDtypeStruct((N, 16), jnp.float32),
            jax.ShapeDtypeStruct((N, 16), jnp.float32),
            jax.ShapeDtypeStruct((N, 16), jnp.float32),
            jax.ShapeDtypeStruct((N, 16), jnp.float32),
            jax.ShapeDtypeStruct((1, 1, 16), jnp.float32),
        ],
        scratch_shapes=[pltpu.VMEM((1, 128), jnp.float32)],
    )
    q2, k2, v2, s2, mv2 = dense2(
        pm1, pd1, s1, Wq2, Wk2, Wv2, Ws2,
        bq2.reshape(1, 16), bk2.reshape(1, 16),
        bv2.reshape(1, 16), bs2.reshape(1, 16),
        _SELQ2, _SELK2, _REP, _T12, _T22)

    pm2, pd2 = _edge_call_16(q2, k2, v2, src_p, dst_p, mv2.reshape(16))

    final = pl.pallas_call(
        _final_body,
        grid=(GRID,),
        in_specs=[
            pl.BlockSpec((NC, BN, 16), lambda i: (0, i, 0)),
            pl.BlockSpec((NC, BN, 16), lambda i: (0, i, 0)),
            _row_spec(16),
        ],
        out_specs=_row_spec(16),
        out_shape=jax.ShapeDtypeStruct((N, 16), jnp.float32),
    )
    return final(pm2, pd2, s2)


# speedup vs baseline: 1.0744x; 1.0744x over previous
import functools
import numpy as np
import jax
import jax.numpy as jnp
from jax import lax
from jax.experimental import pallas as pl
from jax.experimental.pallas import tpu as pltpu
from jax.experimental.pallas import tpu_sc as plsc

N = 10000
E = 320000
D = 128
NC = 2
NS = 16
NW = NC * NS
EPW = E // NW
NSP = 10112
ZPT = 632
ZR = 320
RPT = 624


def _gather16(x, idx):
    dn = lax.GatherDimensionNumbers(
        offset_dims=(), collapsed_slice_dims=(0,), start_index_map=(0,))
    return lax.gather(x, idx[:, None], dn, (1,),
                      mode=lax.GatherScatterMode.PROMISE_IN_BOUNDS)


def _lane_perm(x, stride):
    return _gather16(x, lax.iota(jnp.int32, 16) ^ stride)


def _group_sum(t, width):
    s = 1
    while s < width:
        t = t + _lane_perm(t, s)
        s *= 2
    return t


def _make_edge_call(F, group, EB):
    VPR = F // 16
    NCHUNK = EPW // EB

    mesh = plsc.VectorSubcoreMesh(core_axis_name="c", subcore_axis_name="s")

    @functools.partial(
        pl.kernel,
        out_type=(pltpu.HBM((NC, N, F), jnp.float32),
                  pltpu.HBM((NC, N, 16), jnp.float32)),
        mesh=mesh,
        compiler_params=pltpu.CompilerParams(use_tc_tiling_on_sc=False),
        scratch_types=[
            pltpu.VMEM((EB,), jnp.int32),
            pltpu.VMEM((EB,), jnp.int32),
            pltpu.VMEM((EB,), jnp.int32),
            pltpu.VMEM((EB,), jnp.int32),
            pltpu.VMEM((EB, F), jnp.float32),
            pltpu.VMEM((EB, F), jnp.float32),
            pltpu.VMEM((EB, F), jnp.float32),
            pltpu.VMEM((EB, F), jnp.float32),
            pltpu.VMEM((EB, F), jnp.float32),
            pltpu.VMEM((EB, F), jnp.float32),
            pltpu.VMEM((EB, F), jnp.float32),
            pltpu.VMEM((EB, F), jnp.float32),
            pltpu.VMEM((EB, 16), jnp.float32),
            pltpu.VMEM((EB, 16), jnp.float32),
            pltpu.VMEM((EB,), jnp.int32),
            pltpu.VMEM((EB,), jnp.int32),
            pltpu.VMEM((ZR, F), jnp.float32),
            pltpu.VMEM((ZR, 16), jnp.float32),
            pltpu.VMEM((F,), jnp.float32),
            pltpu.SemaphoreType.DMA,
            pltpu.SemaphoreType.DMA,
            pltpu.SemaphoreType.DMA,
            pltpu.VMEM_SHARED((NSP, F), jnp.float32),
            pltpu.VMEM_SHARED((NSP, 16), jnp.float32),
        ],
    )
    def edge_kernel(q_hbm, k_hbm, v_hbm, src_hbm, dst_hbm, mv_hbm,
                    outm_hbm, outd_hbm,
                    sv0, sv1, dv0, dv1, qv0, qv1, kv0, kv1, vv0, vv1,
                    mb0, mb1, db0, db1, sc0, sc1,
                    zbuf, zbufd, mv, sem_i, sem_r, sem_s,
                    accm_sh, accd_sh):
        svs, dvs = (sv0, sv1), (dv0, dv1)
        qvs, kvs, vvs = (qv0, qv1), (kv0, kv1), (vv0, vv1)
        mbufs, dbufs, scs = (mb0, mb1), (db0, db1), (sc0, sc1)
        HB = EB // 2

        def split_gather(tab, idxr, dstbuf):
            pltpu.async_copy(tab.at[idxr.at[pl.ds(0, HB)]],
                             dstbuf.at[pl.ds(0, HB)], sem_r)
            pltpu.async_copy(tab.at[idxr.at[pl.ds(HB, HB)]],
                             dstbuf.at[pl.ds(HB, HB)], sem_r)
        cid = lax.axis_index("c")
        sid = lax.axis_index("s")
        wid = sid * NC + cid

        zvec = jnp.zeros((16,), jnp.float32)
        lane = lax.iota(jnp.int32, 16)

        def zrow(i, _):
            for j in range(F // 16):
                zbuf[i, pl.ds(16 * j, 16)] = zvec
            zbufd[i, pl.ds(0, 16)] = zvec
            return 0
        lax.fori_loop(0, ZR, zrow, 0)

        r0 = sid * ZPT
        pltpu.sync_copy(zbuf.at[pl.ds(0, 312)], accm_sh.at[pl.ds(r0, 312)])
        pltpu.sync_copy(zbuf, accm_sh.at[pl.ds(r0 + 312, 320)])
        pltpu.sync_copy(zbufd.at[pl.ds(0, 312)], accd_sh.at[pl.ds(r0, 312)])
        pltpu.sync_copy(zbufd, accd_sh.at[pl.ds(r0 + 312, 320)])
        pltpu.sync_copy(mv_hbm, mv)
        plsc.subcore_barrier()

        base = wid * EPW

        mvj = [mv[pl.ds(16 * j, 16)] for j in range(VPR)]
        pick_idx = (lane & 1) * 8
        hsels = [(lane >> 1) == j for j in range(VPR)]
        lane0 = lane == 0

        def body(c, b, do_idx, do_gather, drain_scat=True):
            pltpu.make_async_copy(q_hbm.at[pl.ds(0, EB)], qvs[b],
                                  sem_r).wait()
            pltpu.make_async_copy(k_hbm.at[pl.ds(0, EB)], kvs[b],
                                  sem_r).wait()
            pltpu.make_async_copy(v_hbm.at[pl.ds(0, EB)], vvs[b],
                                  sem_r).wait()
            if drain_scat:
                pltpu.make_async_copy(mbufs[b], accm_sh.at[pl.ds(0, EB)],
                                      sem_s).wait()
                pltpu.make_async_copy(dbufs[b], accd_sh.at[pl.ds(0, EB)],
                                      sem_s).wait()
            nb = 1 - b
            if do_gather:
                pltpu.make_async_copy(src_hbm.at[pl.ds(0, EB)], svs[nb],
                                      sem_i).wait()
                pltpu.make_async_copy(dst_hbm.at[pl.ds(0, EB)], dvs[nb],
                                      sem_i).wait()
                split_gather(q_hbm, dvs[nb], qvs[nb])
                split_gather(k_hbm, svs[nb], kvs[nb])
                split_gather(v_hbm, svs[nb], vvs[nb])

            off = base + c * EB
            qv, kv, vv = qvs[b], kvs[b], vvs[b]
            mbuf, dbuf = mbufs[b], dbufs[b]

            def edge_body(e2, _):
                for ee in range(2):
                    e = e2 * 2 + ee
                    den = zvec
                    for j in range(VPR):
                        qr = qv[e, pl.ds(16 * j, 16)]
                        kr = kv[e, pl.ds(16 * j, 16)]
                        vr = vv[e, pl.ds(16 * j, 16)]
                        t = _group_sum(qr * kr, group)
                        ex = jnp.exp(t - mvj[j])
                        mbuf[e, pl.ds(16 * j, 16)] = vr * ex
                        if group == 16:
                            den = jnp.where(lane0, ex, 0.0)
                        else:
                            picked = _gather16(ex, pick_idx)
                            den = den + jnp.where(hsels[j], picked, 0.0)
                    dbuf[e, pl.ds(0, 16)] = den
                return 0
            lax.fori_loop(0, EB // 2, edge_body, 0)

            def cpidx(i, _):
                scs[b][pl.ds(i * 16, 16)] = dvs[b][pl.ds(i * 16, 16)]
                return 0
            lax.fori_loop(0, EB // 16, cpidx, 0)

            pltpu.async_copy(mbuf, accm_sh.at[scs[b]], sem_s, add=True)
            pltpu.async_copy(dbuf, accd_sh.at[scs[b]], sem_s, add=True)

            if do_idx:
                off2 = base + (c + 2) * EB
                pltpu.async_copy(src_hbm.at[pl.ds(off2, EB)], svs[b], sem_i)
                pltpu.async_copy(dst_hbm.at[pl.ds(off2, EB)], dvs[b], sem_i)

        pltpu.sync_copy(src_hbm.at[pl.ds(base, EB)], sv0)
        pltpu.sync_copy(dst_hbm.at[pl.ds(base, EB)], dv0)
        pltpu.async_copy(src_hbm.at[pl.ds(base + EB, EB)], sv1, sem_i)
        pltpu.async_copy(dst_hbm.at[pl.ds(base + EB, EB)], dv1, sem_i)
        split_gather(q_hbm, dv0, qv0)
        split_gather(k_hbm, sv0, kv0)
        split_gather(v_hbm, sv0, vv0)

        def pair_body(c2, _):
            c = c2 * 2
            body(c, 0, True, True)
            body(c + 1, 1, True, True)
            return 0
        body(0, 0, True, True, drain_scat=False)
        body(1, 1, True, True, drain_scat=False)
        if NCHUNK % 2 == 0:
            lax.fori_loop(1, NCHUNK // 2 - 1, pair_body, 0)
        else:
            lax.fori_loop(1, (NCHUNK - 3) // 2, pair_body, 0)
            body(NCHUNK - 3, 0, True, True)
        body(NCHUNK - 2, (NCHUNK - 2) % 2, False, True)
        body(NCHUNK - 1, (NCHUNK - 1) % 2, False, False)

        for bb in range(2):
            pltpu.make_async_copy(mbufs[bb], accm_sh.at[pl.ds(0, EB)],
                                  sem_s).wait()
            pltpu.make_async_copy(dbufs[bb], accd_sh.at[pl.ds(0, EB)],
                                  sem_s).wait()

        plsc.subcore_barrier()
        w0 = sid * RPT
        pltpu.sync_copy(accm_sh.at[pl.ds(w0, RPT)],
                        outm_hbm.at[cid, pl.ds(w0, RPT)])
        pltpu.sync_copy(accd_sh.at[pl.ds(w0, RPT)],
                        outd_hbm.at[cid, pl.ds(w0, RPT)])

        @pl.when(sid == NS - 1)
        def _tail():
            pltpu.sync_copy(accm_sh.at[pl.ds(NS * RPT, N - NS * RPT)],
                            outm_hbm.at[cid, pl.ds(NS * RPT, N - NS * RPT)])
            pltpu.sync_copy(accd_sh.at[pl.ds(NS * RPT, N - NS * RPT)],
                            outd_hbm.at[cid, pl.ds(NS * RPT, N - NS * RPT)])

    return edge_kernel


_edge_call_64 = _make_edge_call(64, 8, 80)
_edge_call_16 = _make_edge_call(16, 16, 80)

BN = 1000
GRID = N // BN


def _dense1_body(x_ref, wq_ref, wk_ref, wv_ref, ws_ref,
                 bq_ref, bk_ref, bv_ref, bs_ref, selq_ref, selk_ref,
                 t1_ref, t2_ref,
                 q_ref, k_ref, v_ref, s_ref, mv_ref, mn_scr):
    i = pl.program_id(0)
    xb = x_ref[...]
    dn = (((1,), (1,)), ((), ()))
    q = (lax.dot_general(xb, wq_ref[...], dn,
                         preferred_element_type=jnp.float32)
         + bq_ref[...]) * np.float32(1.0 / np.sqrt(8.0))
    k = lax.dot_general(xb, wk_ref[...], dn,
                        preferred_element_type=jnp.float32) + bk_ref[...]
    v = lax.dot_general(xb, wv_ref[...], dn,
                        preferred_element_type=jnp.float32) + bv_ref[...]
    s = lax.dot_general(xb, ws_ref[...], dn,
                        preferred_element_type=jnp.float32) + bs_ref[...]
    q_ref[...] = q
    k_ref[...] = k
    v_ref[...] = v
    s_ref[...] = s
    dn2 = (((1,), (0,)), ((), ()))
    qn = lax.dot_general(q * q, selq_ref[...], dn2,
                         preferred_element_type=jnp.float32)
    kn = lax.dot_general(k * k, selk_ref[...], dn2,
                         preferred_element_type=jnp.float32)
    cur = (jnp.max(qn, axis=0, keepdims=True)
           + jnp.max(kn, axis=0, keepdims=True))

    @pl.when(i == 0)
    def _init():
        mn_scr[...] = cur

    @pl.when(i > 0)
    def _acc():
        mn_scr[...] = jnp.maximum(mn_scr[...], cur)

    @pl.when(i == GRID - 1)
    def _fin():
        mn = mn_scr[...]
        mv_ref[...] = jnp.sqrt(
            lax.dot_general(mn, t1_ref[...], dn2,
                            preferred_element_type=jnp.float32)
            * lax.dot_general(mn, t2_ref[...], dn2,
                              preferred_element_type=jnp.float32))[None]


def _dense2_body(pm_ref, pd_ref, s1_ref, wq_ref, wk_ref, wv_ref, ws_ref,
                 bq_ref, bk_ref, bv_ref, bs_ref, selq_ref, selk_ref, rep_ref,
                 t1_ref, t2_ref,
                 q_ref, k_ref, v_ref, s_ref, mv_ref, mn_scr):
    i = pl.program_id(0)
    pm = pm_ref[...]
    pd = pd_ref[...]
    acc = pm[0] + pm[1]
    den8 = pd[0] + pd[1]
    dn2 = (((1,), (0,)), ((), ()))
    den = lax.dot_general(den8, rep_ref[...], dn2,
                          preferred_element_type=jnp.float32)
    o = acc / (den + 1e-16) + s1_ref[...]
    h = jnp.where(o > 0, o, jnp.exp(o) - 1.0)
    dn = (((1,), (1,)), ((), ()))
    q = (lax.dot_general(h, wq_ref[...], dn,
                         preferred_element_type=jnp.float32)
         + bq_ref[...]) * np.float32(0.25)
    k = lax.dot_general(h, wk_ref[...], dn,
                        preferred_element_type=jnp.float32) + bk_ref[...]
    v = lax.dot_general(h, wv_ref[...], dn,
                        preferred_element_type=jnp.float32) + bv_ref[...]
    s = lax.dot_general(h, ws_ref[...], dn,
                        preferred_element_type=jnp.float32) + bs_ref[...]
    q_ref[...] = q
    k_ref[...] = k
    v_ref[...] = v
    s_ref[...] = s
    qn = lax.dot_general(q * q, selq_ref[...], dn2,
                         preferred_element_type=jnp.float32)
    kn = lax.dot_general(k * k, selk_ref[...], dn2,
                         preferred_element_type=jnp.float32)
    cur = (jnp.max(qn, axis=0, keepdims=True)
           + jnp.max(kn, axis=0, keepdims=True))

    @pl.when(i == 0)
    def _init():
        mn_scr[...] = cur

    @pl.when(i > 0)
    def _acc():
        mn_scr[...] = jnp.maximum(mn_scr[...], cur)

    @pl.when(i == GRID - 1)
    def _fin():
        mn = mn_scr[...]
        mv_ref[...] = jnp.sqrt(
            lax.dot_general(mn, t1_ref[...], dn2,
                            preferred_element_type=jnp.float32)
            * lax.dot_general(mn, t2_ref[...], dn2,
                              preferred_element_type=jnp.float32))[None]


def _final_body(pm_ref, pd_ref, s2_ref, o_ref):
    pm = pm_ref[...]
    pd = pd_ref[...]
    acc = pm[0] + pm[1]
    den = (pd[0] + pd[1])[:, 0:1]
    o = acc / (den + 1e-16) + s2_ref[...]
    m = jnp.max(o, axis=1, keepdims=True)
    ex = jnp.exp(o - m)
    o_ref[...] = o - m - jnp.log(jnp.sum(ex, axis=1, keepdims=True))


def _sel_mats(F, heads):
    c = F // heads
    sq = np.zeros((F, 128), np.float32)
    sk = np.zeros((F, 128), np.float32)
    for h in range(heads):
        sq[h * c:(h + 1) * c, h] = 1.0
        sk[h * c:(h + 1) * c, heads + h] = 1.0
    return sq, sk


_SELQ1, _SELK1 = _sel_mats(64, 8)
_SELQ2, _SELK2 = _sel_mats(16, 1)


def _t_mats(F, heads):
    c = F // heads
    t1 = np.zeros((128, F), np.float32)
    t2 = np.zeros((128, F), np.float32)
    for h in range(heads):
        t1[h, h * c:(h + 1) * c] = 1.0
        t2[heads + h, h * c:(h + 1) * c] = 1.0
    return t1, t2


_T11, _T21 = _t_mats(64, 8)
_T12, _T22 = _t_mats(16, 1)

_REP = np.zeros((16, 64), np.float32)
for _h in range(8):
    _REP[_h, _h * 8:(_h + 1) * 8] = 1.0


def _row_spec(w):
    return pl.BlockSpec((BN, w), lambda i: (i, 0))


def _full_spec(shape):
    nd = len(shape)
    return pl.BlockSpec(shape, lambda i, _n=nd: (0,) * _n)


def kernel(x, edge_index, Wq1, bq1, Wk1, bk1, Wv1, bv1, Ws1, bs1,
           Wq2, bq2, Wk2, bk2, Wv2, bv2, Ws2, bs2):
    src_p = edge_index[0].astype(jnp.int32)
    dst_p = edge_index[1].astype(jnp.int32)

    dense1 = pl.pallas_call(
        _dense1_body,
        grid=(GRID,),
        in_specs=[
            _row_spec(D),
            _full_spec((64, D)), _full_spec((64, D)),
            _full_spec((64, D)), _full_spec((64, D)),
            _full_spec((1, 64)), _full_spec((1, 64)),
            _full_spec((1, 64)), _full_spec((1, 64)),
            _full_spec((64, 128)), _full_spec((64, 128)),
            _full_spec((128, 64)), _full_spec((128, 64)),
        ],
        out_specs=[
            _row_spec(64), _row_spec(64), _row_spec(64), _row_spec(64),
            pl.BlockSpec((1, 1, 64), lambda i: (0, 0, 0)),
        ],
        out_shape=[
            jax.ShapeDtypeStruct((N, 64), jnp.float32),
            jax.ShapeDtypeStruct((N, 64), jnp.float32),
            jax.ShapeDtypeStruct((N, 64), jnp.float32),
            jax.ShapeDtypeStruct((N, 64), jnp.float32),
            jax.ShapeDtypeStruct((1, 1, 64), jnp.float32),
        ],
        scratch_shapes=[pltpu.VMEM((1, 128), jnp.float32)],
    )
    q1, k1, v1, s1, mv1 = dense1(
        x, Wq1, Wk1, Wv1, Ws1,
        bq1.reshape(1, 64), bk1.reshape(1, 64),
        bv1.reshape(1, 64), bs1.reshape(1, 64),
        _SELQ1, _SELK1, _T11, _T21)

    pm1, pd1 = _edge_call_64(q1, k1, v1, src_p, dst_p, mv1.reshape(64))

    dense2 = pl.pallas_call(
        _dense2_body,
        grid=(GRID,),
        in_specs=[
            pl.BlockSpec((NC, BN, 64), lambda i: (0, i, 0)),
            pl.BlockSpec((NC, BN, 16), lambda i: (0, i, 0)),
            _row_spec(64),
            _full_spec((16, 64)), _full_spec((16, 64)),
            _full_spec((16, 64)), _full_spec((16, 64)),
            _full_spec((1, 16)), _full_spec((1, 16)),
            _full_spec((1, 16)), _full_spec((1, 16)),
            _full_spec((16, 128)), _full_spec((16, 128)), _full_spec((16, 64)),
            _full_spec((128, 16)), _full_spec((128, 16)),
        ],
        out_specs=[
            _row_spec(16), _row_spec(16), _row_spec(16), _row_spec(16),
            pl.BlockSpec((1, 1, 16), lambda i: (0, 0, 0)),
        ],
        out_shape=[
            jax.ShapeDtypeStruct((N, 16), jnp.float32),
            jax.ShapeDtypeStruct((N, 16), jnp.float32),
            jax.ShapeDtypeStruct((N, 16), jnp.float32),
            jax.ShapeDtypeStruct((N, 16), jnp.float32),
            jax.ShapeDtypeStruct((1, 1, 16), jnp.float32),
        ],
        scratch_shapes=[pltpu.VMEM((1, 128), jnp.float32)],
    )
    q2, k2, v2, s2, mv2 = dense2(
        pm1, pd1, s1, Wq2, Wk2, Wv2, Ws2,
        bq2.reshape(1, 16), bk2.reshape(1, 16),
        bv2.reshape(1, 16), bs2.reshape(1, 16),
        _SELQ2, _SELK2, _REP, _T12, _T22)

    pm2, pd2 = _edge_call_16(q2, k2, v2, src_p, dst_p, mv2.reshape(16))

    final = pl.pallas_call(
        _final_body,
        grid=(GRID,),
        in_specs=[
            pl.BlockSpec((NC, BN, 16), lambda i: (0, i, 0)),
            pl.BlockSpec((NC, BN, 16), lambda i: (0, i, 0)),
            _row_spec(16),
        ],
        out_specs=_row_spec(16),
        out_shape=jax.ShapeDtypeStruct((N, 16), jnp.float32),
    )
    return final(pm2, pd2, s2)
